# baseline (speedup 1.0000x reference)
import jax
import jax.numpy as jnp
from jax.experimental import pallas as pl
from jax.experimental.pallas import tpu as pltpu

IN_F = 10
OUT_F = 5
GROUP = 64
BR = 1024


def _round_up(n: int, m: int) -> int:
    return ((n + m - 1) // m) * m


def _packed_linear_kernel(x_ref, w_ref, b_ref, o_ref):
    acc = jnp.dot(x_ref[...], w_ref[...], preferred_element_type=jnp.float32)
    o_ref[...] = (acc + b_ref[...]).astype(o_ref.dtype)


@jax.jit
def _forward(x, w_packed, b_packed):
    B, in_f = x.shape
    assert in_f == IN_F

    w = w_packed[:, :OUT_F]
    b = b_packed[:, :OUT_F]
    w_big = jnp.kron(jnp.eye(GROUP, dtype=x.dtype), w)
    b_big = jnp.tile(b, (1, GROUP))

    b_pad = _round_up(B, GROUP * BR)
    xp = jnp.pad(x, ((0, b_pad - B), (0, 0))) if b_pad != B else x
    rows = b_pad // GROUP
    x4 = xp.reshape(rows, GROUP * IN_F)

    y4 = pl.pallas_call(
        _packed_linear_kernel,
        out_shape=jax.ShapeDtypeStruct((rows, GROUP * OUT_F), x.dtype),
        grid=(rows // BR,),
        in_specs=[
            pl.BlockSpec((BR, GROUP * IN_F), lambda i: (i, 0)),
            pl.BlockSpec((GROUP * IN_F, GROUP * OUT_F), lambda i: (0, 0)),
            pl.BlockSpec((1, GROUP * OUT_F), lambda i: (0, 0)),
        ],
        out_specs=pl.BlockSpec((BR, GROUP * OUT_F), lambda i: (i, 0)),
        compiler_params=pltpu.CompilerParams(
            dimension_semantics=("parallel",),
        ),
    )(x4, w_big, b_big)

    y = y4.reshape(b_pad, OUT_F)
    return y[:B] if b_pad != B else y


def kernel(x, w_packed, b_packed):
    return _forward(x, w_packed, b_packed)
